# baseline (speedup 1.0000x reference)
import jax
import jax.numpy as jnp
from jax import lax
from jax.experimental import pallas as pl
from jax.experimental.pallas import tpu as pltpu
from jax.experimental.pallas import tpu_sc as plsc

NC = 2
NS = 16
L = 16
NW = NC * NS

BATCH = 16384
NF = 32
NV = 1000000
BPW = BATCH // NW
NWAVE = BPW // L


def _gmf_body(u_idx_hbm, i_idx_hbm, ut_hbm, it_hbm, par_hbm, out_hbm,
              idx_u, idx_i, stage, packed_u, packed_i, out_v, par_v,
              sem_u, sem_i):
  wid = lax.axis_index("s") * NC + lax.axis_index("c")
  base = pl.multiple_of(wid * BPW, BPW)

  pltpu.sync_copy(u_idx_hbm.at[wid], idx_u)
  pltpu.sync_copy(i_idx_hbm.at[wid], idx_i)
  pltpu.sync_copy(par_hbm, par_v)

  iota = lax.iota(jnp.int32, L)

  def drain(sem):
    pltpu.make_async_copy(ut_hbm.at[pl.ds(0, NF), pl.ds(0, L * 128)],
                          stage, sem).wait()

  def wave(w, idx, tab_hbm, packed, sem):
    wb = pl.multiple_of(w * L, L)
    iv = idx[pl.ds(wb, L)]
    cvec = iv & 127
    for k in range(L):
      ab = pl.multiple_of((iv[k] >> 7) * 128, 128)
      for tb in range(4):
        pltpu.async_copy(tab_hbm.at[pl.ds(tb * 8, 8), pl.ds(ab, 128)],
                         stage.at[k, pl.ds(tb * 8, 8)], sem)
    drain(sem)
    for f in range(NF):
      vals = plsc.load_gather(stage, [iota, jnp.full((L,), f, jnp.int32),
                                      cvec])
      packed[f, pl.ds(wb, L)] = vals

  def wave_body(w, _):
    wave(w, idx_u, ut_hbm, packed_u, sem_u)
    wave(w, idx_i, it_hbm, packed_i, sem_i)
    return 0

  lax.fori_loop(0, NWAVE, wave_body, 0)

  bias = par_v[pl.ds(NF, L)]
  wv0 = par_v[pl.ds(0, L)]
  wv1 = par_v[pl.ds(L, L)]
  w_s = [wv0[k] for k in range(L)] + [wv1[k] for k in range(L)]

  def group_body(g, _):
    col = pl.multiple_of(g * L, L)
    acc = jnp.zeros((L,), jnp.float32)
    for f in range(NF):
      uv = packed_u[f, pl.ds(col, L)]
      iv = packed_i[f, pl.ds(col, L)]
      acc = acc + uv * iv * w_s[f]
    x = acc + bias
    res = 5.0 / (1.0 + jnp.exp(-x))
    out_v[pl.ds(col, L)] = res
    return 0

  lax.fori_loop(0, NWAVE, group_body, 0)

  pltpu.sync_copy(out_v, out_hbm.at[pl.ds(base, BPW)])


@jax.jit
def _gmf(u_idx, i_idx, ut_t, it_t, params):
  mesh = plsc.VectorSubcoreMesh(core_axis_name="c", subcore_axis_name="s")
  run = pl.kernel(
      _gmf_body,
      out_type=jax.ShapeDtypeStruct((BATCH,), jnp.float32),
      mesh=mesh,
      compiler_params=pltpu.CompilerParams(needs_layout_passes=False),
      scratch_types=[
          pltpu.VMEM((BPW,), jnp.int32),
          pltpu.VMEM((BPW,), jnp.int32),
          pltpu.VMEM((L, NF, 128), jnp.float32),
          pltpu.VMEM((NF, BPW), jnp.float32),
          pltpu.VMEM((NF, BPW), jnp.float32),
          pltpu.VMEM((BPW,), jnp.float32),
          pltpu.VMEM((NF + L,), jnp.float32),
          pltpu.SemaphoreType.DMA,
          pltpu.SemaphoreType.DMA,
      ],
  )
  return run(u_idx, i_idx, ut_t, it_t, params)


def kernel(users, items, user_table, item_table, linear_w, linear_b):
  u_idx = (users - 1).astype(jnp.int32).reshape(NW, BPW)
  i_idx = (items - 1).astype(jnp.int32).reshape(NW, BPW)
  ut_t = user_table.T
  it_t = item_table.T
  params = jnp.concatenate(
      [linear_w.reshape(-1), jnp.broadcast_to(linear_b, (L,))]
  ).astype(jnp.float32)
  return _gmf(u_idx, i_idx, ut_t, it_t, params)


# speedup vs baseline: 1.0079x; 1.0079x over previous
import jax
import jax.numpy as jnp
from jax import lax
from jax.experimental import pallas as pl
from jax.experimental.pallas import tpu as pltpu
from jax.experimental.pallas import tpu_sc as plsc

NC = 2
NS = 16
L = 16
NW = NC * NS

BATCH = 16384
NF = 32
NV = 1000000
BPW = BATCH // NW
NWAVE = BPW // L


def _gmf_body(u_idx_hbm, i_idx_hbm, ut_hbm, it_hbm, par_hbm, out_hbm,
              idx_u, idx_i, stage, packed_u, packed_i, out_v, par_v,
              sem_u, sem_i):
  wid = lax.axis_index("s") * NC + lax.axis_index("c")
  base = pl.multiple_of(wid * BPW, BPW)

  pltpu.sync_copy(u_idx_hbm.at[wid], idx_u)
  pltpu.sync_copy(i_idx_hbm.at[wid], idx_i)
  pltpu.sync_copy(par_hbm, par_v)

  iota = lax.iota(jnp.int32, L)

  def drain(sem):
    pltpu.make_async_copy(ut_hbm.at[pl.ds(0, NF), pl.ds(0, L * 128)],
                          stage, sem).wait()

  def wave(w, idx, tab_hbm, packed, sem):
    wb = pl.multiple_of(w * L, L)
    iv = idx[pl.ds(wb, L)]
    cvec = iv & 127
    for k in range(L):
      ab = pl.multiple_of((iv[k] >> 7) * 128, 128)
      pltpu.async_copy(tab_hbm.at[pl.ds(0, NF), pl.ds(ab, 128)],
                       stage.at[k], sem)
    drain(sem)
    for f in range(NF):
      vals = plsc.load_gather(stage, [iota, jnp.full((L,), f, jnp.int32),
                                      cvec])
      packed[f, pl.ds(wb, L)] = vals

  def wave_body(w, _):
    wave(w, idx_u, ut_hbm, packed_u, sem_u)
    wave(w, idx_i, it_hbm, packed_i, sem_i)
    return 0

  lax.fori_loop(0, NWAVE, wave_body, 0)

  bias = par_v[pl.ds(NF, L)]
  wv0 = par_v[pl.ds(0, L)]
  wv1 = par_v[pl.ds(L, L)]
  w_s = [wv0[k] for k in range(L)] + [wv1[k] for k in range(L)]

  def group_body(g, _):
    col = pl.multiple_of(g * L, L)
    acc = jnp.zeros((L,), jnp.float32)
    for f in range(NF):
      uv = packed_u[f, pl.ds(col, L)]
      iv = packed_i[f, pl.ds(col, L)]
      acc = acc + uv * iv * w_s[f]
    x = acc + bias
    res = 5.0 / (1.0 + jnp.exp(-x))
    out_v[pl.ds(col, L)] = res
    return 0

  lax.fori_loop(0, NWAVE, group_body, 0)

  pltpu.sync_copy(out_v, out_hbm.at[pl.ds(base, BPW)])


@jax.jit
def _gmf(u_idx, i_idx, ut_t, it_t, params):
  mesh = plsc.VectorSubcoreMesh(core_axis_name="c", subcore_axis_name="s")
  run = pl.kernel(
      _gmf_body,
      out_type=jax.ShapeDtypeStruct((BATCH,), jnp.float32),
      mesh=mesh,
      compiler_params=pltpu.CompilerParams(needs_layout_passes=False),
      scratch_types=[
          pltpu.VMEM((BPW,), jnp.int32),
          pltpu.VMEM((BPW,), jnp.int32),
          pltpu.VMEM((L, NF, 128), jnp.float32),
          pltpu.VMEM((NF, BPW), jnp.float32),
          pltpu.VMEM((NF, BPW), jnp.float32),
          pltpu.VMEM((BPW,), jnp.float32),
          pltpu.VMEM((NF + L,), jnp.float32),
          pltpu.SemaphoreType.DMA,
          pltpu.SemaphoreType.DMA,
      ],
  )
  return run(u_idx, i_idx, ut_t, it_t, params)


def kernel(users, items, user_table, item_table, linear_w, linear_b):
  u_idx = (users - 1).astype(jnp.int32).reshape(NW, BPW)
  i_idx = (items - 1).astype(jnp.int32).reshape(NW, BPW)
  ut_t = user_table.T
  it_t = item_table.T
  params = jnp.concatenate(
      [linear_w.reshape(-1), jnp.broadcast_to(linear_b, (L,))]
  ).astype(jnp.float32)
  return _gmf(u_idx, i_idx, ut_t, it_t, params)


# speedup vs baseline: 1.0105x; 1.0025x over previous
import jax
import jax.numpy as jnp
from jax import lax
from jax.experimental import pallas as pl
from jax.experimental.pallas import tpu as pltpu
from jax.experimental.pallas import tpu_sc as plsc

NC = 2
NS = 16
L = 16
NW = NC * NS

BATCH = 16384
NF = 32
BPW = BATCH // NW
NWAVE = BPW // L


def _gmf_body(u_idx_hbm, i_idx_hbm, ut_hbm, it_hbm, par_hbm, out_hbm,
              idx_u, idx_i, stage, packed_u, packed_i, out_v, par_v,
              sem_u, sem_i):
  wid = lax.axis_index("s") * NC + lax.axis_index("c")
  base = pl.multiple_of(wid * BPW, BPW)

  pltpu.sync_copy(u_idx_hbm.at[wid], idx_u)
  pltpu.sync_copy(i_idx_hbm.at[wid], idx_i)
  pltpu.sync_copy(par_hbm, par_v)

  iota = lax.iota(jnp.int32, L)

  def drain(sem):
    pltpu.make_async_copy(ut_hbm.at[pl.ds(0, NF), pl.ds(0, L * 128)],
                          stage, sem).wait()

  def wave(w, idx, tab_hbm, packed, sem):
    wb = pl.multiple_of(w * L, L)
    iv = idx[pl.ds(wb, L)]
    cvec = iv & 127
    for k in range(L):
      ab = pl.multiple_of((iv[k] >> 7) * 128, 128)
      pltpu.async_copy(tab_hbm.at[pl.ds(0, NF), pl.ds(ab, 128)],
                       stage.at[k], sem)
    drain(sem)
    for f in range(NF):
      vals = plsc.load_gather(stage, [iota, jnp.full((L,), f, jnp.int32),
                                      cvec])
      packed[f, pl.ds(wb, L)] = vals

  def wave_body(w, _):
    wave(w, idx_u, ut_hbm, packed_u, sem_u)
    wave(w, idx_i, it_hbm, packed_i, sem_i)
    return 0

  lax.fori_loop(0, NWAVE, wave_body, 0)

  bias = par_v[pl.ds(NF, L)]
  wv0 = par_v[pl.ds(0, L)]
  wv1 = par_v[pl.ds(L, L)]
  w_s = [wv0[k] for k in range(L)] + [wv1[k] for k in range(L)]

  def group_body(g, _):
    col = pl.multiple_of(g * L, L)
    acc = jnp.zeros((L,), jnp.float32)
    for f in range(NF):
      uv = packed_u[f, pl.ds(col, L)]
      iv = packed_i[f, pl.ds(col, L)]
      acc = acc + uv * iv * w_s[f]
    x = acc + bias
    res = 5.0 / (1.0 + jnp.exp(-x))
    out_v[pl.ds(col, L)] = res
    return 0

  lax.fori_loop(0, NWAVE, group_body, 0)

  pltpu.sync_copy(out_v, out_hbm.at[pl.ds(base, BPW)])


@jax.jit
def _gmf(u_idx, i_idx, ut_t, it_t, params):
  mesh = plsc.VectorSubcoreMesh(core_axis_name="c", subcore_axis_name="s")
  run = pl.kernel(
      _gmf_body,
      out_type=jax.ShapeDtypeStruct((BATCH,), jnp.float32),
      mesh=mesh,
      compiler_params=pltpu.CompilerParams(needs_layout_passes=False),
      scratch_types=[
          pltpu.VMEM((BPW,), jnp.int32),
          pltpu.VMEM((BPW,), jnp.int32),
          pltpu.VMEM((L, NF, 128), jnp.float32),
          pltpu.VMEM((NF, BPW), jnp.float32),
          pltpu.VMEM((NF, BPW), jnp.float32),
          pltpu.VMEM((BPW,), jnp.float32),
          pltpu.VMEM((NF + L,), jnp.float32),
          pltpu.SemaphoreType.DMA,
          pltpu.SemaphoreType.DMA,
      ],
  )
  return run(u_idx, i_idx, ut_t, it_t, params)


def kernel(users, items, user_table, item_table, linear_w, linear_b):
  u_idx = (users - 1).astype(jnp.int32).reshape(NW, BPW)
  i_idx = (items - 1).astype(jnp.int32).reshape(NW, BPW)
  ut_t = user_table.T
  it_t = item_table.T
  params = jnp.concatenate(
      [linear_w.reshape(-1), jnp.broadcast_to(linear_b, (L,))]
  ).astype(jnp.float32)
  return _gmf(u_idx, i_idx, ut_t, it_t, params)
